# baseline (speedup 1.0000x reference)
import functools

import jax
import jax.numpy as jnp
from jax import lax
from jax.experimental import pallas as pl
from jax.experimental.pallas import tpu as pltpu
from jax.experimental.pallas import tpu_sc as plsc

_B, _L = 16384, 50
_N = _B * _L
_NW = 32
_PW = _N // _NW
_NCHUNK = 2
_CW = _PW // _NCHUNK
_VPC = _CW // 16
_ACC = 80

_mesh = plsc.VectorSubcoreMesh(core_axis_name="c", subcore_axis_name="s")


@functools.partial(
    pl.kernel,
    out_type=jax.ShapeDtypeStruct((_NW * _ACC,), jnp.float32),
    mesh=_mesh,
    scratch_types=[
        pltpu.VMEM((_CW + 16,), jnp.int32),
        pltpu.VMEM((_CW,), jnp.int32),
        pltpu.VMEM((_CW,), jnp.float32),
        pltpu.VMEM((_CW,), jnp.float32),
        pltpu.VMEM((_CW,), jnp.float32),
        pltpu.VMEM((_CW,), jnp.float32),
        pltpu.VMEM((_ACC,), jnp.float32),
    ],
)
def _rc_kernel(seq_hbm, fg_hbm, slp_hbm, bnl_hbm, fgl_hbm, r_hbm, out_hbm,
               seq_v, fg_v, slp_v, bnl_v, fgl_v, r_v, acc_v):
    wid = lax.axis_index("s") * 2 + lax.axis_index("c")
    base = wid * _PW
    lane = lax.iota(jnp.int32, 16)
    zero = jnp.zeros((16,), jnp.float32)
    one = jnp.ones((16,), jnp.float32)
    accs = (zero, zero, zero, zero, zero)

    for c in range(_NCHUNK):
        off = base + c * _CW
        pltpu.sync_copy(seq_hbm.at[pl.ds(off, _CW)], seq_v.at[pl.ds(16, _CW)])
        pltpu.sync_copy(fg_hbm.at[pl.ds(off, _CW)], fg_v)
        pltpu.sync_copy(slp_hbm.at[pl.ds(off, _CW)], slp_v)
        pltpu.sync_copy(bnl_hbm.at[pl.ds(off, _CW)], bnl_v)
        pltpu.sync_copy(fgl_hbm.at[pl.ds(off, _CW)], fgl_v)
        pltpu.sync_copy(r_hbm.at[pl.ds(off, _CW)], r_v)

        def vec_body(t, accs):
            s1, c1, s2, s3, c2 = accs
            k0 = t * 16
            kvec = lane + k0
            first = lax.rem(kvec, jnp.int32(_L)) == 0
            sprev = seq_v[pl.ds(15 + k0, 16)]
            smask = jnp.logical_or(first, sprev > 0)
            slp = slp_v[pl.ds(k0, 16)]
            r = r_v[pl.ds(k0, 16)]
            bnl = bnl_v[pl.ds(k0, 16)]
            fgl = fgl_v[pl.ds(k0, 16)]
            bmask = fg_v[pl.ds(k0, 16)] > 0
            s1 = s1 + jnp.where(smask, slp * r, zero)
            c1 = c1 + jnp.where(smask, one, zero)
            s2 = s2 + jnp.where(bmask, bnl * r, zero)
            s3 = s3 + jnp.where(bmask, fgl * r, zero)
            c2 = c2 + jnp.where(bmask, one, zero)
            return (s1, c1, s2, s3, c2)

        accs = lax.fori_loop(0, _VPC, vec_body, accs)

    for i, acc in enumerate(accs):
        acc_v[pl.ds(16 * i, 16)] = acc
    pltpu.sync_copy(acc_v, out_hbm.at[pl.ds(wid * _ACC, _ACC)])


def kernel(seq, bn_seq, fg_seq, seqLogprobs, bnLogprobs, fgLogprobs, reward):
    del bn_seq
    parts = _rc_kernel(
        seq.reshape(-1).astype(jnp.int32),
        fg_seq.reshape(-1).astype(jnp.int32),
        seqLogprobs.reshape(-1),
        bnLogprobs.reshape(-1),
        fgLogprobs.reshape(-1),
        reward.reshape(-1),
    )
    p = parts.reshape(_NW, 5, 16).sum(axis=(0, 2))
    cnt = jnp.maximum(p[4], 1.0)
    return (-p[0] / p[1], -p[2] / cnt, -p[3] / cnt)


# speedup vs baseline: 1.0777x; 1.0777x over previous
import functools

import jax
import jax.numpy as jnp
from jax import lax
from jax.experimental import pallas as pl
from jax.experimental.pallas import tpu as pltpu
from jax.experimental.pallas import tpu_sc as plsc

_B, _L = 16384, 50
_N = _B * _L
_NW = 32
_PW = _N // _NW
_NCHUNK = 4
_CW = _PW // _NCHUNK
_VPC = _CW // 16
_ACC = 80

_mesh = plsc.VectorSubcoreMesh(core_axis_name="c", subcore_axis_name="s")


def _chunk_bufs():
    return [
        pltpu.VMEM((_CW + 16,), jnp.int32),
        pltpu.VMEM((_CW,), jnp.int32),
        pltpu.VMEM((_CW,), jnp.float32),
        pltpu.VMEM((_CW,), jnp.float32),
        pltpu.VMEM((_CW,), jnp.float32),
        pltpu.VMEM((_CW,), jnp.float32),
    ]


@functools.partial(
    pl.kernel,
    out_type=jax.ShapeDtypeStruct((_NW * _ACC,), jnp.float32),
    mesh=_mesh,
    scratch_types=_chunk_bufs() + _chunk_bufs() + [
        pltpu.VMEM((_ACC,), jnp.float32),
        pltpu.SemaphoreType.DMA,
        pltpu.SemaphoreType.DMA,
    ],
)
def _rc_kernel(seq_hbm, fg_hbm, slp_hbm, bnl_hbm, fgl_hbm, r_hbm, out_hbm,
               seq0, fg0, slp0, bnl0, fgl0, r0,
               seq1, fg1, slp1, bnl1, fgl1, r1,
               acc_v, sem0, sem1):
    wid = lax.axis_index("s") * 2 + lax.axis_index("c")
    base = wid * _PW
    hbms = (seq_hbm, fg_hbm, slp_hbm, bnl_hbm, fgl_hbm, r_hbm)
    bufs = ((seq0, fg0, slp0, bnl0, fgl0, r0),
            (seq1, fg1, slp1, bnl1, fgl1, r1))
    sems = (sem0, sem1)
    lane = lax.iota(jnp.int32, 16)
    zero = jnp.zeros((16,), jnp.float32)
    one = jnp.ones((16,), jnp.float32)

    def fire(c, s):
        off = base + c * _CW
        bs = bufs[s]
        hs = [pltpu.async_copy(seq_hbm.at[pl.ds(off, _CW)],
                               bs[0].at[pl.ds(16, _CW)], sems[s])]
        for hbm, b in zip(hbms[1:], bs[1:]):
            hs.append(pltpu.async_copy(hbm.at[pl.ds(off, _CW)], b, sems[s]))
        return hs

    def compute(bs, accs):
        seq_v, fg_v, slp_v, bnl_v, fgl_v, r_v = bs

        def body(t, accs):
            s1, c1, s2, s3, c2 = accs
            k0 = t * 16
            kvec = lane + k0
            first = lax.rem(kvec, jnp.int32(_L)) == 0
            sprev = seq_v[pl.ds(15 + k0, 16)]
            smask = jnp.logical_or(first, sprev > 0)
            slp = slp_v[pl.ds(k0, 16)]
            r = r_v[pl.ds(k0, 16)]
            bnl = bnl_v[pl.ds(k0, 16)]
            fgl = fgl_v[pl.ds(k0, 16)]
            bmask = fg_v[pl.ds(k0, 16)] > 0
            s1 = s1 + jnp.where(smask, slp * r, zero)
            c1 = c1 + jnp.where(smask, one, zero)
            s2 = s2 + jnp.where(bmask, bnl * r, zero)
            s3 = s3 + jnp.where(bmask, fgl * r, zero)
            c2 = c2 + jnp.where(bmask, one, zero)
            return (s1, c1, s2, s3, c2)

        return plsc.parallel_loop(0, _VPC, unroll=8, carry=accs)(body)

    accs = (zero, zero, zero, zero, zero)
    pend = fire(0, 0)
    for c in range(_NCHUNK):
        s = c % 2
        nxt = fire(c + 1, 1 - s) if c + 1 < _NCHUNK else None
        for h in pend:
            h.wait()
        pend = nxt
        accs = compute(bufs[s], accs)

    for i, acc in enumerate(accs):
        acc_v[pl.ds(16 * i, 16)] = acc
    pltpu.sync_copy(acc_v, out_hbm.at[pl.ds(wid * _ACC, _ACC)])


def kernel(seq, bn_seq, fg_seq, seqLogprobs, bnLogprobs, fgLogprobs, reward):
    del bn_seq
    parts = _rc_kernel(
        seq.reshape(-1).astype(jnp.int32),
        fg_seq.reshape(-1).astype(jnp.int32),
        seqLogprobs.reshape(-1),
        bnLogprobs.reshape(-1),
        fgLogprobs.reshape(-1),
        reward.reshape(-1),
    )
    p = parts.reshape(_NW, 5, 16).sum(axis=(0, 2))
    cnt = jnp.maximum(p[4], 1.0)
    return (-p[0] / p[1], -p[2] / cnt, -p[3] / cnt)


# speedup vs baseline: 1.2209x; 1.1329x over previous
import functools

import jax
import jax.numpy as jnp
from jax import lax
from jax.experimental import pallas as pl
from jax.experimental.pallas import tpu as pltpu

_B, _L = 16384, 50
_ROWS = _B * _L // 128
_BR = 400
_GRID = _ROWS // _BR


def _body(seq_ref, fg_ref, slp_ref, bnl_ref, fgl_ref, r_ref, acc_ref):
    b = pl.program_id(0)
    shape = (_BR, 128)
    row = lax.broadcasted_iota(jnp.int32, shape, 0)
    lane = lax.broadcasted_iota(jnp.int32, shape, 1)

    seq = seq_ref[...]
    seq_l = pltpu.roll(seq, 1, 1)
    prev = jnp.where(lane == 0, pltpu.roll(seq_l, 1, 0), seq_l)
    k = (b * _BR + row) * 128 + lane
    first = lax.rem(k, jnp.int32(_L)) == 0
    smask = jnp.logical_or(first, prev > 0)

    r = r_ref[...]
    x = slp_ref[...] * r
    bmask = fg_ref[...] > 0
    zero = jnp.zeros(shape, jnp.float32)
    one = jnp.ones(shape, jnp.float32)

    def red(v):
        return jnp.sum(v.reshape(_BR // 8, 8, 128), axis=0)

    p = (red(jnp.where(smask, x, zero)),
         red(jnp.where(smask, one, zero)),
         red(jnp.where(bmask, bnl_ref[...] * r, zero)),
         red(jnp.where(bmask, fgl_ref[...] * r, zero)),
         red(jnp.where(bmask, one, zero)))

    @pl.when(b == 0)
    def _():
        for i in range(5):
            acc_ref[i] = p[i]

    @pl.when(b > 0)
    def _():
        for i in range(5):
            acc_ref[i] += p[i]


_in_spec = pl.BlockSpec((_BR, 128), lambda b: (b, 0))

_call = pl.pallas_call(
    _body,
    grid=(_GRID,),
    in_specs=[_in_spec] * 6,
    out_specs=pl.BlockSpec((5, 8, 128), lambda b: (0, 0, 0)),
    out_shape=jax.ShapeDtypeStruct((5, 8, 128), jnp.float32),
    compiler_params=pltpu.CompilerParams(
        dimension_semantics=("arbitrary",),
    ),
)


def kernel(seq, bn_seq, fg_seq, seqLogprobs, bnLogprobs, fgLogprobs, reward):
    del bn_seq
    acc = _call(
        seq.reshape(_ROWS, 128).astype(jnp.int32),
        fg_seq.reshape(_ROWS, 128).astype(jnp.int32),
        seqLogprobs.reshape(_ROWS, 128),
        bnLogprobs.reshape(_ROWS, 128),
        fgLogprobs.reshape(_ROWS, 128),
        reward.reshape(_ROWS, 128),
    )
    p = acc.sum(axis=(1, 2))
    cnt = jnp.maximum(p[4], 1.0)
    return (-p[0] / p[1], -p[2] / cnt, -p[3] / cnt)


# speedup vs baseline: 2.1799x; 1.7854x over previous
import jax
import jax.numpy as jnp
from jax import lax
from jax.experimental import pallas as pl
from jax.experimental.pallas import tpu as pltpu

_B, _L = 16384, 50
_BR = 1024
_GRID = _B // _BR


def _body(seq_ref, fg_ref, slp_ref, bnl_ref, fgl_ref, r_ref, acc_ref):
    b = pl.program_id(0)
    shape = (_BR, _L)
    lane = lax.broadcasted_iota(jnp.int32, shape, 1)

    prev = pltpu.roll(seq_ref[...], 1, 1)
    smask = jnp.logical_or(lane == 0, prev > 0)

    r = r_ref[...]
    x = slp_ref[...] * r
    bmask = fg_ref[...] > 0
    zero = jnp.zeros(shape, jnp.float32)
    one = jnp.ones(shape, jnp.float32)

    def red(v):
        return jnp.sum(v.reshape(_BR // 8, 8, _L), axis=0)

    p = (red(jnp.where(smask, x, zero)),
         red(jnp.where(smask, one, zero)),
         red(jnp.where(bmask, bnl_ref[...] * r, zero)),
         red(jnp.where(bmask, fgl_ref[...] * r, zero)),
         red(jnp.where(bmask, one, zero)))

    @pl.when(b == 0)
    def _():
        for i in range(5):
            acc_ref[i] = p[i]

    @pl.when(b > 0)
    def _():
        for i in range(5):
            acc_ref[i] += p[i]


_in_spec = pl.BlockSpec((_BR, _L), lambda b: (b, 0))

_call = pl.pallas_call(
    _body,
    grid=(_GRID,),
    in_specs=[_in_spec] * 6,
    out_specs=pl.BlockSpec((5, 8, _L), lambda b: (0, 0, 0)),
    out_shape=jax.ShapeDtypeStruct((5, 8, _L), jnp.float32),
    compiler_params=pltpu.CompilerParams(
        dimension_semantics=("arbitrary",),
    ),
)


def kernel(seq, bn_seq, fg_seq, seqLogprobs, bnLogprobs, fgLogprobs, reward):
    del bn_seq
    acc = _call(
        seq.astype(jnp.int32),
        fg_seq.astype(jnp.int32),
        seqLogprobs,
        bnLogprobs,
        fgLogprobs,
        reward,
    )
    p = acc.sum(axis=(1, 2))
    cnt = jnp.maximum(p[4], 1.0)
    return (-p[0] / p[1], -p[2] / cnt, -p[3] / cnt)


# speedup vs baseline: 2.2789x; 1.0454x over previous
import jax
import jax.numpy as jnp
from jax import lax
from jax.experimental import pallas as pl
from jax.experimental.pallas import tpu as pltpu

_B, _L = 16384, 50
_BR = 2048
_GRID = _B // _BR


def _body(seq_ref, fg_ref, slp_ref, bnl_ref, fgl_ref, r_ref, acc_ref):
    b = pl.program_id(0)
    shape = (_BR, _L)
    lane = lax.broadcasted_iota(jnp.int32, shape, 1)

    prev = pltpu.roll(seq_ref[...], 1, 1)
    smask = jnp.logical_or(lane == 0, prev > 0)

    r = r_ref[...]
    x = slp_ref[...] * r
    bmask = fg_ref[...] > 0
    zero = jnp.zeros(shape, jnp.float32)
    one = jnp.ones(shape, jnp.float32)

    def red(v):
        return jnp.sum(v.reshape(_BR // 8, 8, _L), axis=0)

    p = (red(jnp.where(smask, x, zero)),
         red(jnp.where(smask, one, zero)),
         red(jnp.where(bmask, bnl_ref[...] * r, zero)),
         red(jnp.where(bmask, fgl_ref[...] * r, zero)),
         red(jnp.where(bmask, one, zero)))

    @pl.when(b == 0)
    def _():
        for i in range(5):
            acc_ref[i] = p[i]

    @pl.when(b > 0)
    def _():
        for i in range(5):
            acc_ref[i] += p[i]


_in_spec = pl.BlockSpec((_BR, _L), lambda b: (b, 0))

_call = pl.pallas_call(
    _body,
    grid=(_GRID,),
    in_specs=[_in_spec] * 6,
    out_specs=pl.BlockSpec((5, 8, _L), lambda b: (0, 0, 0)),
    out_shape=jax.ShapeDtypeStruct((5, 8, _L), jnp.float32),
    compiler_params=pltpu.CompilerParams(
        dimension_semantics=("arbitrary",),
    ),
)


def kernel(seq, bn_seq, fg_seq, seqLogprobs, bnLogprobs, fgLogprobs, reward):
    del bn_seq
    acc = _call(
        seq.astype(jnp.int32),
        fg_seq.astype(jnp.int32),
        seqLogprobs,
        bnLogprobs,
        fgLogprobs,
        reward,
    )
    p = acc.sum(axis=(1, 2))
    cnt = jnp.maximum(p[4], 1.0)
    return (-p[0] / p[1], -p[2] / cnt, -p[3] / cnt)


# speedup vs baseline: 2.3007x; 1.0096x over previous
import jax
import jax.numpy as jnp
from jax import lax
from jax.experimental import pallas as pl
from jax.experimental.pallas import tpu as pltpu

_B, _L = 16384, 50
_GRID = 8
_NSPLIT = 4
_BR = _B // (_GRID * _NSPLIT)


def _body(*refs):
    acc_ref = refs[-1]
    b = pl.program_id(0)
    shape = (_BR, _L)
    lane = lax.broadcasted_iota(jnp.int32, shape, 1)
    zero = jnp.zeros(shape, jnp.float32)
    one = jnp.ones(shape, jnp.float32)

    def red(v):
        return jnp.sum(v.reshape(_BR // 8, 8, _L), axis=0)

    p = [jnp.zeros((8, _L), jnp.float32) for _ in range(5)]
    for j in range(_NSPLIT):
        seq_ref, fg_ref, slp_ref, bnl_ref, fgl_ref, r_ref = (
            refs[0 * _NSPLIT + j], refs[1 * _NSPLIT + j],
            refs[2 * _NSPLIT + j], refs[3 * _NSPLIT + j],
            refs[4 * _NSPLIT + j], refs[5 * _NSPLIT + j])
        prev = pltpu.roll(seq_ref[...], 1, 1)
        smask = jnp.logical_or(lane == 0, prev > 0)
        r = r_ref[...]
        x = slp_ref[...] * r
        bmask = fg_ref[...] > 0
        p[0] += red(jnp.where(smask, x, zero))
        p[1] += red(jnp.where(smask, one, zero))
        p[2] += red(jnp.where(bmask, bnl_ref[...] * r, zero))
        p[3] += red(jnp.where(bmask, fgl_ref[...] * r, zero))
        p[4] += red(jnp.where(bmask, one, zero))

    @pl.when(b == 0)
    def _():
        for i in range(5):
            acc_ref[i] = p[i]

    @pl.when(b > 0)
    def _():
        for i in range(5):
            acc_ref[i] += p[i]


def _spec(j):
    return pl.BlockSpec((_BR, _L), lambda b, jj=j: (_NSPLIT * b + jj, 0))


_call = pl.pallas_call(
    _body,
    grid=(_GRID,),
    in_specs=[_spec(j) for _ in range(6) for j in range(_NSPLIT)],
    out_specs=pl.BlockSpec((5, 8, _L), lambda b: (0, 0, 0)),
    out_shape=jax.ShapeDtypeStruct((5, 8, _L), jnp.float32),
    compiler_params=pltpu.CompilerParams(
        dimension_semantics=("arbitrary",),
    ),
)


def kernel(seq, bn_seq, fg_seq, seqLogprobs, bnLogprobs, fgLogprobs, reward):
    del bn_seq
    arrs = (seq.astype(jnp.int32), fg_seq.astype(jnp.int32),
            seqLogprobs, bnLogprobs, fgLogprobs, reward)
    acc = _call(*[a for a in arrs for _ in range(_NSPLIT)])
    p = acc.sum(axis=(1, 2))
    cnt = jnp.maximum(p[4], 1.0)
    return (-p[0] / p[1], -p[2] / cnt, -p[3] / cnt)


# speedup vs baseline: 7.1152x; 3.0926x over previous
import jax
import jax.numpy as jnp
from jax import lax
from jax.experimental import pallas as pl
from jax.experimental.pallas import tpu as pltpu

_B, _L = 16384, 50
_BC = 2048
_GRID = _B // _BC


def _body(seq_ref, fg_ref, slp_ref, bnl_ref, fgl_ref, r_ref, acc_ref):
    b = pl.program_id(0)
    shape = (_L, _BC)
    row = lax.broadcasted_iota(jnp.int32, shape, 0)

    prev = pltpu.roll(seq_ref[...], 1, 0)
    smask = jnp.logical_or(row == 0, prev > 0)

    r = r_ref[...]
    x = slp_ref[...] * r
    bmask = fg_ref[...] > 0
    zero = jnp.zeros(shape, jnp.float32)
    one = jnp.ones(shape, jnp.float32)

    p = (jnp.sum(jnp.where(smask, x, zero), axis=0),
         jnp.sum(jnp.where(smask, one, zero), axis=0),
         jnp.sum(jnp.where(bmask, bnl_ref[...] * r, zero), axis=0),
         jnp.sum(jnp.where(bmask, fgl_ref[...] * r, zero), axis=0),
         jnp.sum(jnp.where(bmask, one, zero), axis=0))

    @pl.when(b == 0)
    def _():
        for i in range(5):
            acc_ref[i] = p[i]

    @pl.when(b > 0)
    def _():
        for i in range(5):
            acc_ref[i] += p[i]


_in_spec = pl.BlockSpec((_L, _BC), lambda b: (0, b))

_call = pl.pallas_call(
    _body,
    grid=(_GRID,),
    in_specs=[_in_spec] * 6,
    out_specs=pl.BlockSpec((5, _BC), lambda b: (0, 0)),
    out_shape=jax.ShapeDtypeStruct((5, _BC), jnp.float32),
    compiler_params=pltpu.CompilerParams(
        dimension_semantics=("arbitrary",),
    ),
)


def kernel(seq, bn_seq, fg_seq, seqLogprobs, bnLogprobs, fgLogprobs, reward):
    del bn_seq
    acc = _call(
        seq.T.astype(jnp.int32),
        fg_seq.T.astype(jnp.int32),
        seqLogprobs.T,
        bnLogprobs.T,
        fgLogprobs.T,
        reward.T,
    )
    p = acc.sum(axis=1)
    cnt = jnp.maximum(p[4], 1.0)
    return (-p[0] / p[1], -p[2] / cnt, -p[3] / cnt)


# speedup vs baseline: 10.6444x; 1.4960x over previous
import jax
import jax.numpy as jnp
from jax import lax
from jax.experimental import pallas as pl
from jax.experimental.pallas import tpu as pltpu

_B, _L = 16384, 50
_BC = 2048
_GRID = _B // _BC


def _body(seq_ref, fg_ref, slp_ref, bnl_ref, fgl_ref, r_ref, out_ref, acc_ref):
    b = pl.program_id(0)
    shape = (_L, _BC)
    row = lax.broadcasted_iota(jnp.int32, shape, 0)

    prev = pltpu.roll(seq_ref[...], 1, 0)
    smask = jnp.logical_or(row == 0, prev > 0)

    r = r_ref[...]
    x = slp_ref[...] * r
    bmask = fg_ref[...] > 0
    zero = jnp.zeros(shape, jnp.float32)
    one = jnp.ones(shape, jnp.float32)

    p = (jnp.sum(jnp.where(smask, x, zero), axis=0),
         jnp.sum(jnp.where(smask, one, zero), axis=0),
         jnp.sum(jnp.where(bmask, bnl_ref[...] * r, zero), axis=0),
         jnp.sum(jnp.where(bmask, fgl_ref[...] * r, zero), axis=0),
         jnp.sum(jnp.where(bmask, one, zero), axis=0))

    @pl.when(b == 0)
    def _():
        for i in range(5):
            acc_ref[i] = p[i]

    @pl.when(b > 0)
    def _():
        for i in range(5):
            acc_ref[i] += p[i]

    @pl.when(b == _GRID - 1)
    def _():
        s1 = jnp.sum(acc_ref[0])
        c1 = jnp.sum(acc_ref[1])
        s2 = jnp.sum(acc_ref[2])
        s3 = jnp.sum(acc_ref[3])
        c2 = jnp.maximum(jnp.sum(acc_ref[4]), 1.0)
        lane = lax.broadcasted_iota(jnp.int32, (1, 128), 1)
        out_ref[...] = jnp.where(
            lane == 0, -s1 / c1, jnp.where(lane == 1, -s2 / c2, -s3 / c2))


_in_spec = pl.BlockSpec((_L, _BC), lambda b: (0, b))

_call = pl.pallas_call(
    _body,
    grid=(_GRID,),
    in_specs=[_in_spec] * 6,
    out_specs=pl.BlockSpec((1, 128), lambda b: (0, 0)),
    out_shape=jax.ShapeDtypeStruct((1, 128), jnp.float32),
    scratch_shapes=[pltpu.VMEM((5, _BC), jnp.float32)],
    compiler_params=pltpu.CompilerParams(
        dimension_semantics=("arbitrary",),
    ),
)


def kernel(seq, bn_seq, fg_seq, seqLogprobs, bnLogprobs, fgLogprobs, reward):
    del bn_seq
    out = _call(
        seq.T.astype(jnp.int32),
        fg_seq.T.astype(jnp.int32),
        seqLogprobs.T,
        bnLogprobs.T,
        fgLogprobs.T,
        reward.T,
    )
    return (out[0, 0], out[0, 1], out[0, 2])
